# baseline (speedup 1.0000x reference)
import jax
import jax.numpy as jnp
from jax.experimental import pallas as pl
from jax.experimental.pallas import tpu as pltpu

B = 8192
D = 768
L = 256
K = 8192

RB = 512
KB = 512
NR = B // RB
NK = K // KB


def _dkn_body(x_ref, ax_ref, h_ref, c_ref, recon_ref, cl_ref, min_scr):
    i = pl.program_id(0)
    k = pl.program_id(1)

    @pl.when((k == 0) & (i == 0))
    def _init_out():
        recon_ref[...] = jnp.zeros_like(recon_ref)
        cl_ref[...] = jnp.zeros_like(cl_ref)

    @pl.when(k == 0)
    def _recon():
        d = x_ref[...] - ax_ref[...]
        recon_ref[...] += jnp.sum(d * d)

    h = h_ref[...]
    c = c_ref[...]
    cb = c.astype(jnp.bfloat16)
    ones = jnp.ones((1, L), jnp.bfloat16)
    c2 = jax.lax.dot_general(ones, cb * cb, (((1,), (1,)), ((), ())),
                             preferred_element_type=jnp.float32)
    hc = jax.lax.dot_general(h.astype(jnp.bfloat16), cb,
                             (((1,), (1,)), ((), ())),
                             preferred_element_type=jnp.float32)
    part = jnp.min(c2 - 2.0 * hc, axis=1, keepdims=True)

    @pl.when(k == 0)
    def _min_init():
        min_scr[...] = part

    @pl.when(k > 0)
    def _min_acc():
        min_scr[...] = jnp.minimum(min_scr[...], part)

    @pl.when(k == NK - 1)
    def _cl_final():
        h2 = jnp.sum(h * h, axis=1, keepdims=True)
        d2min = jnp.maximum(h2 + min_scr[...], 0.0)
        cl_ref[...] += jnp.sum(d2min)


def kernel(x, h_x, a_x, cluster_centers):
    recon_sum, cl_sum = pl.pallas_call(
        _dkn_body,
        grid=(NR, NK),
        in_specs=[
            pl.BlockSpec((RB, D), lambda i, k: (i, 0)),
            pl.BlockSpec((RB, D), lambda i, k: (i, 0)),
            pl.BlockSpec((RB, L), lambda i, k: (i, 0)),
            pl.BlockSpec((KB, L), lambda i, k: (k, 0)),
        ],
        out_specs=[
            pl.BlockSpec((1, 1), lambda i, k: (0, 0)),
            pl.BlockSpec((1, 1), lambda i, k: (0, 0)),
        ],
        out_shape=[
            jax.ShapeDtypeStruct((1, 1), jnp.float32),
            jax.ShapeDtypeStruct((1, 1), jnp.float32),
        ],
        scratch_shapes=[pltpu.VMEM((RB, 1), jnp.float32)],
    )(x, a_x, h_x, cluster_centers)
    return (recon_sum[0, 0] / (B * D)) + (cl_sum[0, 0] / (B * L))
